# baseline (speedup 1.0000x reference)
import functools

import jax
import jax.numpy as jnp
from jax import lax
from jax.experimental import pallas as pl
from jax.experimental.pallas import tpu as pltpu
from jax.experimental.pallas import tpu_sc as plsc

_N = 10000
_E = 320000
_D = 128
_H = 128
_C = 64

_NC = 2
_NS = 16
_NW = _NC * _NS
_K = 80
_DEGW = 16

_NPAD = 10240
_PER_TILE = _E // _NW
_STEPS = _PER_TILE // _K
_ROWS_PER_TILE = _NPAD // _NS

_mesh = functools.partial(
    plsc.VectorSubcoreMesh, core_axis_name="c", subcore_axis_name="s",
    num_cores=_NC, num_subcores=_NS)


_DBUF = 4


@functools.cache
def _get_sc_degree():
  @functools.partial(
      pl.kernel,
      out_type=jax.ShapeDtypeStruct((_NC, _NPAD, _DEGW), jnp.float32),
      mesh=_mesh(),
      scratch_types=(
          [pltpu.VMEM((_K,), jnp.int32) for _ in range(_DBUF)]
          + [pltpu.VMEM((_K, _DEGW), jnp.float32)]
          + [pltpu.SemaphoreType.DMA for _ in range(_DBUF)]
          + [pltpu.VMEM_SHARED((_NPAD, _DEGW), jnp.float32)]
      ),
      compiler_params=pltpu.CompilerParams(use_tc_tiling_on_sc=False),
  )
  def _sc_degree(dst_hbm, ones_hbm, zeros_hbm, out_hbm, *scr):
    idx = scr[:_DBUF]
    ones_v = scr[_DBUF]
    ssem = scr[_DBUF + 1:2 * _DBUF + 1]
    acc = scr[2 * _DBUF + 1]
    cid = lax.axis_index("c")
    sid = lax.axis_index("s")
    r0 = sid * _ROWS_PER_TILE
    pltpu.sync_copy(zeros_hbm, acc.at[pl.ds(r0, _ROWS_PER_TILE)])
    pltpu.sync_copy(ones_hbm, ones_v)
    plsc.subcore_barrier()
    base = (cid * _NS + sid) * _PER_TILE

    def fire(b, c):
        pltpu.sync_copy(dst_hbm.at[pl.ds(base + c * _K, _K)], idx[b])
        pltpu.async_copy(ones_v, acc.at[idx[b]], add=True, sem=ssem[b])

    def drain(b):
        pltpu.make_async_copy(ones_v, acc.at[idx[b]], ssem[b]).wait()

    for b in range(_DBUF):
        fire(b, b)

    def quad(j, carry):
        for b in range(_DBUF):
            drain(b)
            fire(b, _DBUF * j + _DBUF + b)
        return carry

    lax.fori_loop(0, (_STEPS - 1 - _DBUF) // _DBUF, quad, 0)
    drain(0)
    fire(0, _STEPS - 1)
    for b in range(_DBUF):
        drain(b)
    plsc.subcore_barrier()
    pltpu.sync_copy(acc.at[pl.ds(r0, _ROWS_PER_TILE)],
                    out_hbm.at[cid, pl.ds(r0, _ROWS_PER_TILE)])

  return _sc_degree


_NBUF = 4


@functools.cache
def _get_sc_aggregate():
  @functools.partial(
      pl.kernel,
      out_type=jax.ShapeDtypeStruct((_NC, _NPAD, _H), jnp.float32),
      mesh=_mesh(),
      scratch_types=(
          [pltpu.VMEM((_K,), jnp.int32) for _ in range(2 * _NBUF)]
          + [pltpu.VMEM((_K, _H), jnp.float32) for _ in range(_NBUF)]
          + [pltpu.SemaphoreType.DMA for _ in range(_NBUF)]
          + [pltpu.VMEM_SHARED((_NPAD, _H), jnp.float32)]
      ),
  )
  def _sc_aggregate(u_hbm, src_hbm, dst_hbm, zeros_hbm, out_hbm, *scr):
    sidx = scr[:_NBUF]
    didx = scr[_NBUF:2 * _NBUF]
    rows = scr[2 * _NBUF:3 * _NBUF]
    gsem = scr[3 * _NBUF:4 * _NBUF]
    acc = scr[4 * _NBUF]
    cid = lax.axis_index("c")
    sid = lax.axis_index("s")
    r0 = sid * _ROWS_PER_TILE
    pltpu.sync_copy(zeros_hbm, acc.at[pl.ds(r0, _ROWS_PER_TILE)])
    plsc.subcore_barrier()
    base = (cid * _NS + sid) * _PER_TILE

    def fire(b, off):
        pltpu.sync_copy(src_hbm.at[pl.ds(off, _K)], sidx[b])
        pltpu.sync_copy(dst_hbm.at[pl.ds(off, _K)], didx[b])
        pltpu.async_copy(u_hbm.at[sidx[b]], rows[b], gsem[b])

    def wait_gather(b):
        pltpu.make_async_copy(u_hbm.at[sidx[b]], rows[b], gsem[b]).wait()

    for b in range(_NBUF):
        fire(b, base + b * _K)

    def quad(j, carry):
        for b in range(_NBUF):
            cpre = _NBUF * j + _NBUF + b
            off = base + jnp.where(cpre < _STEPS, cpre, 0) * _K
            wait_gather(b)
            pltpu.sync_copy(rows[b], acc.at[didx[b]], add=True)
            fire(b, off)
        return carry

    lax.fori_loop(0, _STEPS // _NBUF, quad, 0)
    wait_gather(0)
    pltpu.sync_copy(rows[0], acc.at[didx[0]], add=True)
    for b in range(1, _NBUF):
        wait_gather(b)
    plsc.subcore_barrier()
    pltpu.sync_copy(acc.at[pl.ds(r0, _ROWS_PER_TILE)],
                    out_hbm.at[cid, pl.ds(r0, _ROWS_PER_TILE)])

  return _sc_aggregate


_BLK = 1000


def _dinv_block(dp_ref):
    deg = dp_ref[0, :, 0:1] + dp_ref[1, :, 0:1] + 1.0
    return lax.rsqrt(deg)


def _tc1_body(x_ref, w_ref, dp_ref, u_ref):
    dinv = _dinv_block(dp_ref)
    h = jnp.dot(x_ref[...], w_ref[...], preferred_element_type=jnp.float32)
    u_ref[...] = h * dinv


def _tc2_body(a_ref, u_ref, dp_ref, b_ref, w_ref, o_ref):
    dinv = _dinv_block(dp_ref)
    t = dinv * (a_ref[0] + a_ref[1] + u_ref[...]) + b_ref[...]
    h1 = jnp.maximum(t, 0.0)
    o_ref[...] = jnp.dot(
        h1, w_ref[...], preferred_element_type=jnp.float32) * dinv


def _tc3_body(a_ref, u_ref, dp_ref, b_ref, w_ref, bc_ref, o_ref):
    dinv = _dinv_block(dp_ref)
    h2 = dinv * (a_ref[0] + a_ref[1] + u_ref[...]) + b_ref[...]
    logits = jnp.dot(
        h2, w_ref[...], preferred_element_type=jnp.float32) + bc_ref[...]
    m = jnp.max(logits, axis=1, keepdims=True)
    s = logits - m
    lse = jnp.log(jnp.sum(jnp.exp(s), axis=1, keepdims=True))
    o_ref[...] = s - lse


def _row_spec(width):
    return pl.BlockSpec((_BLK, width), lambda i: (i, 0))


_dp_spec = pl.BlockSpec((_NC, _BLK, _DEGW), lambda i: (0, i, 0))


def _tc1(x, W1, deg_parts):
    return pl.pallas_call(
        _tc1_body,
        grid=(_N // _BLK,),
        in_specs=[
            _row_spec(_D),
            pl.BlockSpec((_D, _H), lambda i: (0, 0)),
            _dp_spec,
        ],
        out_specs=_row_spec(_H),
        out_shape=jax.ShapeDtypeStruct((_N, _H), jnp.float32),
    )(x, W1, deg_parts)


def _tc2(agg1, u1, deg_parts, b1, W2):
    return pl.pallas_call(
        _tc2_body,
        grid=(_N // _BLK,),
        in_specs=[
            pl.BlockSpec((_NC, _BLK, _H), lambda i: (0, i, 0)),
            _row_spec(_H),
            _dp_spec,
            pl.BlockSpec((1, _H), lambda i: (0, 0)),
            pl.BlockSpec((_H, _H), lambda i: (0, 0)),
        ],
        out_specs=_row_spec(_H),
        out_shape=jax.ShapeDtypeStruct((_N, _H), jnp.float32),
    )(agg1, u1, deg_parts, b1, W2)


def _tc3(agg2, u2, deg_parts, b2, Wc, bc):
    return pl.pallas_call(
        _tc3_body,
        grid=(_N // _BLK,),
        in_specs=[
            pl.BlockSpec((_NC, _BLK, _H), lambda i: (0, i, 0)),
            _row_spec(_H),
            _dp_spec,
            pl.BlockSpec((1, _H), lambda i: (0, 0)),
            pl.BlockSpec((_H, _C), lambda i: (0, 0)),
            pl.BlockSpec((1, _C), lambda i: (0, 0)),
        ],
        out_specs=_row_spec(_C),
        out_shape=jax.ShapeDtypeStruct((_N, _C), jnp.float32),
    )(agg2, u2, deg_parts, b2, Wc, bc)


def kernel(x, edge_index, W1, b1, W2, b2, Wc, bc):
    src = edge_index[0]
    dst = edge_index[1]
    ones_k = jnp.ones((_K, _DEGW), jnp.float32)
    zeros_deg = jnp.zeros((_ROWS_PER_TILE, _DEGW), jnp.float32)
    zeros_rows = jnp.zeros((_ROWS_PER_TILE, _H), jnp.float32)

    deg_parts = _get_sc_degree()(dst, ones_k, zeros_deg)
    u1 = _tc1(x, W1, deg_parts)
    agg1 = _get_sc_aggregate()(u1, src, dst, zeros_rows)
    u2 = _tc2(agg1, u1, deg_parts, b1.reshape(1, _H), W2)
    agg2 = _get_sc_aggregate()(u2, src, dst, zeros_rows)
    return _tc3(agg2, u2, deg_parts, b2.reshape(1, _H), Wc, bc.reshape(1, _C))


# speedup vs baseline: 1.0134x; 1.0134x over previous
import functools

import jax
import jax.numpy as jnp
from jax import lax
from jax.experimental import pallas as pl
from jax.experimental.pallas import tpu as pltpu
from jax.experimental.pallas import tpu_sc as plsc

_N = 10000
_E = 320000
_D = 128
_H = 128
_C = 64

_NC = 2
_NS = 16
_NW = _NC * _NS
_K = 80
_DEGW = 16

_NPAD = 10240
_PER_TILE = _E // _NW
_STEPS = _PER_TILE // _K
_ROWS_PER_TILE = _NPAD // _NS

_mesh = functools.partial(
    plsc.VectorSubcoreMesh, core_axis_name="c", subcore_axis_name="s",
    num_cores=_NC, num_subcores=_NS)


_DBUF = 4


@functools.cache
def _get_sc_degree():
  @functools.partial(
      pl.kernel,
      out_type=jax.ShapeDtypeStruct((_NC, _NPAD, _DEGW), jnp.float32),
      mesh=_mesh(),
      scratch_types=(
          [pltpu.VMEM((_K,), jnp.int32) for _ in range(_DBUF)]
          + [pltpu.VMEM((_K, _DEGW), jnp.float32)]
          + [pltpu.SemaphoreType.DMA for _ in range(_DBUF)]
          + [pltpu.VMEM_SHARED((_NPAD, _DEGW), jnp.float32)]
      ),
      compiler_params=pltpu.CompilerParams(use_tc_tiling_on_sc=False),
  )
  def _sc_degree(dst_hbm, ones_hbm, zeros_hbm, out_hbm, *scr):
    idx = scr[:_DBUF]
    ones_v = scr[_DBUF]
    ssem = scr[_DBUF + 1:2 * _DBUF + 1]
    acc = scr[2 * _DBUF + 1]
    cid = lax.axis_index("c")
    sid = lax.axis_index("s")
    r0 = sid * _ROWS_PER_TILE
    pltpu.sync_copy(zeros_hbm, acc.at[pl.ds(r0, _ROWS_PER_TILE)])
    pltpu.sync_copy(ones_hbm, ones_v)
    plsc.subcore_barrier()
    base = (cid * _NS + sid) * _PER_TILE

    def fire(b, c):
        pltpu.sync_copy(dst_hbm.at[pl.ds(base + c * _K, _K)], idx[b])
        pltpu.async_copy(ones_v, acc.at[idx[b]], add=True, sem=ssem[b])

    def drain(b):
        pltpu.make_async_copy(ones_v, acc.at[idx[b]], ssem[b]).wait()

    for b in range(_DBUF):
        fire(b, b)

    def quad(j, carry):
        for b in range(_DBUF):
            drain(b)
            fire(b, _DBUF * j + _DBUF + b)
        return carry

    lax.fori_loop(0, (_STEPS - 1 - _DBUF) // _DBUF, quad, 0)
    drain(0)
    fire(0, _STEPS - 1)
    for b in range(_DBUF):
        drain(b)
    plsc.subcore_barrier()
    pltpu.sync_copy(acc.at[pl.ds(r0, _ROWS_PER_TILE)],
                    out_hbm.at[cid, pl.ds(r0, _ROWS_PER_TILE)])

  return _sc_degree


_NBUF = 4


@functools.cache
def _get_sc_aggregate():
  @functools.partial(
      pl.kernel,
      out_type=jax.ShapeDtypeStruct((_NC, _NPAD, _H), jnp.float32),
      mesh=_mesh(),
      scratch_types=(
          [pltpu.VMEM((_K,), jnp.int32) for _ in range(2 * _NBUF)]
          + [pltpu.VMEM((_K, _H), jnp.float32) for _ in range(_NBUF)]
          + [pltpu.SemaphoreType.DMA for _ in range(_NBUF)]
          + [pltpu.VMEM_SHARED((_NPAD, _H), jnp.float32)]
      ),
  )
  def _sc_aggregate(u_hbm, src_hbm, dst_hbm, zeros_hbm, out_hbm, *scr):
    sidx = scr[:_NBUF]
    didx = scr[_NBUF:2 * _NBUF]
    rows = scr[2 * _NBUF:3 * _NBUF]
    gsem = scr[3 * _NBUF:4 * _NBUF]
    acc = scr[4 * _NBUF]
    cid = lax.axis_index("c")
    sid = lax.axis_index("s")
    r0 = sid * _ROWS_PER_TILE
    pltpu.sync_copy(zeros_hbm, acc.at[pl.ds(r0, _ROWS_PER_TILE)])
    plsc.subcore_barrier()
    base = (cid * _NS + sid) * _PER_TILE

    def fire(b, off):
        pltpu.sync_copy(src_hbm.at[pl.ds(off, _K)], sidx[b])
        pltpu.sync_copy(dst_hbm.at[pl.ds(off, _K)], didx[b])
        pltpu.async_copy(u_hbm.at[sidx[b]], rows[b], gsem[b])

    def wait_gather(b):
        pltpu.make_async_copy(u_hbm.at[sidx[b]], rows[b], gsem[b]).wait()

    for b in range(_NBUF):
        fire(b, base + b * _K)

    def quad(j, carry):
        for b in range(_NBUF):
            cpre = _NBUF * j + _NBUF + b
            off = base + jnp.where(cpre < _STEPS, cpre, 0) * _K
            wait_gather(b)
            pltpu.sync_copy(rows[b], acc.at[didx[b]], add=True)
            fire(b, off)
        return carry

    lax.fori_loop(0, _STEPS // _NBUF, quad, 0)
    wait_gather(0)
    pltpu.sync_copy(rows[0], acc.at[didx[0]], add=True)
    for b in range(1, _NBUF):
        wait_gather(b)
    plsc.subcore_barrier()
    pltpu.sync_copy(acc.at[pl.ds(r0, _ROWS_PER_TILE)],
                    out_hbm.at[cid, pl.ds(r0, _ROWS_PER_TILE)])

  return _sc_aggregate


_BLK = 2000


def _dinv_block(dp_ref):
    deg = dp_ref[0, :, 0:1] + dp_ref[1, :, 0:1] + 1.0
    return lax.rsqrt(deg)


def _tc1_body(x_ref, w_ref, dp_ref, u_ref):
    dinv = _dinv_block(dp_ref)
    h = jnp.dot(x_ref[...], w_ref[...], preferred_element_type=jnp.float32)
    u_ref[...] = h * dinv


def _tc2_body(a_ref, u_ref, dp_ref, b_ref, w_ref, o_ref):
    dinv = _dinv_block(dp_ref)
    t = dinv * (a_ref[0] + a_ref[1] + u_ref[...]) + b_ref[...]
    h1 = jnp.maximum(t, 0.0)
    o_ref[...] = jnp.dot(
        h1, w_ref[...], preferred_element_type=jnp.float32) * dinv


def _tc3_body(a_ref, u_ref, dp_ref, b_ref, w_ref, bc_ref, o_ref):
    dinv = _dinv_block(dp_ref)
    h2 = dinv * (a_ref[0] + a_ref[1] + u_ref[...]) + b_ref[...]
    logits = jnp.dot(
        h2, w_ref[...], preferred_element_type=jnp.float32) + bc_ref[...]
    m = jnp.max(logits, axis=1, keepdims=True)
    s = logits - m
    lse = jnp.log(jnp.sum(jnp.exp(s), axis=1, keepdims=True))
    o_ref[...] = s - lse


def _row_spec(width):
    return pl.BlockSpec((_BLK, width), lambda i: (i, 0))


_dp_spec = pl.BlockSpec((_NC, _BLK, _DEGW), lambda i: (0, i, 0))


def _tc1(x, W1, deg_parts):
    return pl.pallas_call(
        _tc1_body,
        grid=(_N // _BLK,),
        in_specs=[
            _row_spec(_D),
            pl.BlockSpec((_D, _H), lambda i: (0, 0)),
            _dp_spec,
        ],
        out_specs=_row_spec(_H),
        out_shape=jax.ShapeDtypeStruct((_N, _H), jnp.float32),
    )(x, W1, deg_parts)


def _tc2(agg1, u1, deg_parts, b1, W2):
    return pl.pallas_call(
        _tc2_body,
        grid=(_N // _BLK,),
        in_specs=[
            pl.BlockSpec((_NC, _BLK, _H), lambda i: (0, i, 0)),
            _row_spec(_H),
            _dp_spec,
            pl.BlockSpec((1, _H), lambda i: (0, 0)),
            pl.BlockSpec((_H, _H), lambda i: (0, 0)),
        ],
        out_specs=_row_spec(_H),
        out_shape=jax.ShapeDtypeStruct((_N, _H), jnp.float32),
    )(agg1, u1, deg_parts, b1, W2)


def _tc3(agg2, u2, deg_parts, b2, Wc, bc):
    return pl.pallas_call(
        _tc3_body,
        grid=(_N // _BLK,),
        in_specs=[
            pl.BlockSpec((_NC, _BLK, _H), lambda i: (0, i, 0)),
            _row_spec(_H),
            _dp_spec,
            pl.BlockSpec((1, _H), lambda i: (0, 0)),
            pl.BlockSpec((_H, _C), lambda i: (0, 0)),
            pl.BlockSpec((1, _C), lambda i: (0, 0)),
        ],
        out_specs=_row_spec(_C),
        out_shape=jax.ShapeDtypeStruct((_N, _C), jnp.float32),
    )(agg2, u2, deg_parts, b2, Wc, bc)


def kernel(x, edge_index, W1, b1, W2, b2, Wc, bc):
    src = edge_index[0]
    dst = edge_index[1]
    ones_k = jnp.ones((_K, _DEGW), jnp.float32)
    zeros_deg = jnp.zeros((_ROWS_PER_TILE, _DEGW), jnp.float32)
    zeros_rows = jnp.zeros((_ROWS_PER_TILE, _H), jnp.float32)

    deg_parts = _get_sc_degree()(dst, ones_k, zeros_deg)
    u1 = _tc1(x, W1, deg_parts)
    agg1 = _get_sc_aggregate()(u1, src, dst, zeros_rows)
    u2 = _tc2(agg1, u1, deg_parts, b1.reshape(1, _H), W2)
    agg2 = _get_sc_aggregate()(u2, src, dst, zeros_rows)
    return _tc3(agg2, u2, deg_parts, b2.reshape(1, _H), Wc, bc.reshape(1, _C))


# speedup vs baseline: 1.2294x; 1.2131x over previous
import functools

import jax
import jax.numpy as jnp
from jax import lax
from jax.experimental import pallas as pl
from jax.experimental.pallas import tpu as pltpu
from jax.experimental.pallas import tpu_sc as plsc

_N = 10000
_E = 320000
_D = 128
_H = 128
_C = 64

_NC = 2
_NS = 16
_NW = _NC * _NS
_K = 80
_DEGW = 16

_NPAD = 10240
_PER_TILE = _E // _NW
_STEPS = _PER_TILE // _K
_ROWS_PER_TILE = _NPAD // _NS

_mesh = functools.partial(
    plsc.VectorSubcoreMesh, core_axis_name="c", subcore_axis_name="s",
    num_cores=_NC, num_subcores=_NS)


_DBUF = 4


@functools.cache
def _get_sc_degree():
  @functools.partial(
      pl.kernel,
      out_type=jax.ShapeDtypeStruct((_NC, _NPAD, _DEGW), jnp.float32),
      mesh=_mesh(),
      scratch_types=(
          [pltpu.VMEM((_K,), jnp.int32) for _ in range(_DBUF)]
          + [pltpu.VMEM((_K, _DEGW), jnp.float32)]
          + [pltpu.SemaphoreType.DMA for _ in range(_DBUF)]
          + [pltpu.VMEM_SHARED((_NPAD, _DEGW), jnp.float32)]
      ),
      compiler_params=pltpu.CompilerParams(use_tc_tiling_on_sc=False),
  )
  def _sc_degree(dst_hbm, ones_hbm, zeros_hbm, out_hbm, *scr):
    idx = scr[:_DBUF]
    ones_v = scr[_DBUF]
    ssem = scr[_DBUF + 1:2 * _DBUF + 1]
    acc = scr[2 * _DBUF + 1]
    cid = lax.axis_index("c")
    sid = lax.axis_index("s")
    r0 = sid * _ROWS_PER_TILE
    pltpu.sync_copy(zeros_hbm, acc.at[pl.ds(r0, _ROWS_PER_TILE)])
    pltpu.sync_copy(ones_hbm, ones_v)
    plsc.subcore_barrier()
    base = (cid * _NS + sid) * _PER_TILE

    def fire(b, c):
        pltpu.sync_copy(dst_hbm.at[pl.ds(base + c * _K, _K)], idx[b])
        pltpu.async_copy(ones_v, acc.at[idx[b]], add=True, sem=ssem[b])

    def drain(b):
        pltpu.make_async_copy(ones_v, acc.at[idx[b]], ssem[b]).wait()

    for b in range(_DBUF):
        fire(b, b)

    def quad(j, carry):
        for b in range(_DBUF):
            drain(b)
            fire(b, _DBUF * j + _DBUF + b)
        return carry

    lax.fori_loop(0, (_STEPS - 1 - _DBUF) // _DBUF, quad, 0)
    drain(0)
    fire(0, _STEPS - 1)
    for b in range(_DBUF):
        drain(b)
    plsc.subcore_barrier()
    pltpu.sync_copy(acc.at[pl.ds(r0, _ROWS_PER_TILE)],
                    out_hbm.at[cid, pl.ds(r0, _ROWS_PER_TILE)])

  return _sc_degree


_NBUF = 4


@functools.cache
def _get_sc_aggregate():
  @functools.partial(
      pl.kernel,
      out_type=jax.ShapeDtypeStruct((_NC, _NPAD, _H), jnp.float32),
      mesh=_mesh(),
      scratch_types=(
          [pltpu.VMEM((2, _K), jnp.int32) for _ in range(_NBUF)]
          + [pltpu.VMEM((_K, _H), jnp.float32) for _ in range(_NBUF)]
          + [pltpu.SemaphoreType.DMA for _ in range(_NBUF)]
          + [pltpu.VMEM_SHARED((_NPAD, _H), jnp.float32)]
      ),
  )
  def _sc_aggregate(u_hbm, idx_hbm, zeros_hbm, out_hbm, *scr):
    idx = scr[:_NBUF]
    rows = scr[_NBUF:2 * _NBUF]
    gsem = scr[2 * _NBUF:3 * _NBUF]
    acc = scr[3 * _NBUF]
    cid = lax.axis_index("c")
    sid = lax.axis_index("s")
    wid = cid * _NS + sid
    r0 = sid * _ROWS_PER_TILE
    pltpu.sync_copy(zeros_hbm, acc.at[pl.ds(r0, _ROWS_PER_TILE)])
    plsc.subcore_barrier()

    def fire(b, c):
        pltpu.sync_copy(idx_hbm.at[wid, c], idx[b])
        pltpu.async_copy(u_hbm.at[idx[b].at[0]], rows[b], gsem[b])

    def wait_gather(b):
        pltpu.make_async_copy(u_hbm.at[idx[b].at[0]], rows[b], gsem[b]).wait()

    for b in range(_NBUF):
        fire(b, b)

    def quad(j, carry):
        for b in range(_NBUF):
            cpre = _NBUF * j + _NBUF + b
            c = jnp.where(cpre < _STEPS, cpre, 0)
            wait_gather(b)
            pltpu.sync_copy(rows[b], acc.at[idx[b].at[1]], add=True)
            fire(b, c)
        return carry

    lax.fori_loop(0, _STEPS // _NBUF, quad, 0)
    wait_gather(0)
    pltpu.sync_copy(rows[0], acc.at[idx[0].at[1]], add=True)
    for b in range(1, _NBUF):
        wait_gather(b)
    plsc.subcore_barrier()
    pltpu.sync_copy(acc.at[pl.ds(r0, _ROWS_PER_TILE)],
                    out_hbm.at[cid, pl.ds(r0, _ROWS_PER_TILE)])

  return _sc_aggregate


_BLK = 2000


def _dinv_block(dp_ref):
    deg = dp_ref[0, :, 0:1] + dp_ref[1, :, 0:1] + 1.0
    return lax.rsqrt(deg)


def _tc1_body(x_ref, w_ref, dp_ref, u_ref):
    dinv = _dinv_block(dp_ref)
    h = jnp.dot(x_ref[...], w_ref[...], preferred_element_type=jnp.float32)
    u_ref[...] = h * dinv


def _tc2_body(a_ref, u_ref, dp_ref, b_ref, w_ref, o_ref):
    dinv = _dinv_block(dp_ref)
    t = dinv * (a_ref[0] + a_ref[1] + u_ref[...]) + b_ref[...]
    h1 = jnp.maximum(t, 0.0)
    o_ref[...] = jnp.dot(
        h1, w_ref[...], preferred_element_type=jnp.float32) * dinv


def _tc3_body(a_ref, u_ref, dp_ref, b_ref, w_ref, bc_ref, o_ref):
    dinv = _dinv_block(dp_ref)
    h2 = dinv * (a_ref[0] + a_ref[1] + u_ref[...]) + b_ref[...]
    logits = jnp.dot(
        h2, w_ref[...], preferred_element_type=jnp.float32) + bc_ref[...]
    m = jnp.max(logits, axis=1, keepdims=True)
    s = logits - m
    lse = jnp.log(jnp.sum(jnp.exp(s), axis=1, keepdims=True))
    o_ref[...] = s - lse


def _row_spec(width):
    return pl.BlockSpec((_BLK, width), lambda i: (i, 0))


_dp_spec = pl.BlockSpec((_NC, _BLK, _DEGW), lambda i: (0, i, 0))


def _tc1(x, W1, deg_parts):
    return pl.pallas_call(
        _tc1_body,
        grid=(_N // _BLK,),
        in_specs=[
            _row_spec(_D),
            pl.BlockSpec((_D, _H), lambda i: (0, 0)),
            _dp_spec,
        ],
        out_specs=_row_spec(_H),
        out_shape=jax.ShapeDtypeStruct((_N, _H), jnp.float32),
    )(x, W1, deg_parts)


def _tc2(agg1, u1, deg_parts, b1, W2):
    return pl.pallas_call(
        _tc2_body,
        grid=(_N // _BLK,),
        in_specs=[
            pl.BlockSpec((_NC, _BLK, _H), lambda i: (0, i, 0)),
            _row_spec(_H),
            _dp_spec,
            pl.BlockSpec((1, _H), lambda i: (0, 0)),
            pl.BlockSpec((_H, _H), lambda i: (0, 0)),
        ],
        out_specs=_row_spec(_H),
        out_shape=jax.ShapeDtypeStruct((_N, _H), jnp.float32),
    )(agg1, u1, deg_parts, b1, W2)


def _tc3(agg2, u2, deg_parts, b2, Wc, bc):
    return pl.pallas_call(
        _tc3_body,
        grid=(_N // _BLK,),
        in_specs=[
            pl.BlockSpec((_NC, _BLK, _H), lambda i: (0, i, 0)),
            _row_spec(_H),
            _dp_spec,
            pl.BlockSpec((1, _H), lambda i: (0, 0)),
            pl.BlockSpec((_H, _C), lambda i: (0, 0)),
            pl.BlockSpec((1, _C), lambda i: (0, 0)),
        ],
        out_specs=_row_spec(_C),
        out_shape=jax.ShapeDtypeStruct((_N, _C), jnp.float32),
    )(agg2, u2, deg_parts, b2, Wc, bc)


def kernel(x, edge_index, W1, b1, W2, b2, Wc, bc):
    src = edge_index[0]
    dst = edge_index[1]
    idx_comb = edge_index.reshape(2, _NW, _STEPS, _K).transpose(1, 2, 0, 3)
    ones_k = jnp.ones((_K, _DEGW), jnp.float32)
    zeros_deg = jnp.zeros((_ROWS_PER_TILE, _DEGW), jnp.float32)
    zeros_rows = jnp.zeros((_ROWS_PER_TILE, _H), jnp.float32)

    deg_parts = _get_sc_degree()(dst, ones_k, zeros_deg)
    u1 = _tc1(x, W1, deg_parts)
    agg1 = _get_sc_aggregate()(u1, idx_comb, zeros_rows)
    u2 = _tc2(agg1, u1, deg_parts, b1.reshape(1, _H), W2)
    agg2 = _get_sc_aggregate()(u2, idx_comb, zeros_rows)
    return _tc3(agg2, u2, deg_parts, b2.reshape(1, _H), Wc, bc.reshape(1, _C))
